# Initial kernel scaffold; baseline (speedup 1.0000x reference)
import jax
import jax.numpy as jnp
from jax.experimental import pallas as pl


def kernel(features, edge_index, edge_weight, W1, b1, W2, b2):
    raise NotImplementedError("write your pallas kernel here")


# speedup vs baseline: 5.4093x; 5.4093x over previous
import functools

import jax
import jax.numpy as jnp
from jax import lax
from jax.experimental import pallas as pl
from jax.experimental.pallas import tpu as pltpu
from jax.experimental.pallas import tpu_sc as plsc

N = 10000
E = 320000
D_IN = 128
DH = 16

NC = 2
NS = 16
NW = NC * NS

NPAD = 10112
ROWS_PER_SUB = NPAD // NS

CHUNK = 128
CHUNKS_PER_TILE = 79
EP_TILE = CHUNKS_PER_TILE * CHUNK
EPAD = NW * EP_TILE

_MESH = plsc.VectorSubcoreMesh(core_axis_name="c", subcore_axis_name="s")
_SC_PARAMS = pltpu.CompilerParams(use_tc_tiling_on_sc=False)


@functools.partial(
    pl.kernel,
    mesh=_MESH,
    compiler_params=_SC_PARAMS,
    out_type=jax.ShapeDtypeStruct((NC, NPAD, DH), jnp.float32),
    scratch_types=[
        pltpu.VMEM((CHUNK,), jnp.int32),
        pltpu.VMEM((CHUNK,), jnp.int32),
        pltpu.VMEM((CHUNK, DH), jnp.float32),
        pltpu.VMEM((CHUNK, DH), jnp.float32),
        pltpu.VMEM_SHARED((NPAD, DH), jnp.float32),
    ],
)
def _sc_degrees(src_h, dst_h, onerows_h, zeros_h, out_h,
                idxa_v, idxb_v, rowa_v, rowb_v, shared):
    cid = lax.axis_index("c")
    sid = lax.axis_index("s")
    pltpu.sync_copy(zeros_h.at[pl.ds(sid * ROWS_PER_SUB, ROWS_PER_SUB)],
                    shared.at[pl.ds(sid * ROWS_PER_SUB, ROWS_PER_SUB)])
    pltpu.sync_copy(onerows_h.at[0], rowa_v)
    pltpu.sync_copy(onerows_h.at[1], rowb_v)
    plsc.subcore_barrier()

    base = (cid * NS + sid) * EP_TILE

    def body(c, carry):
        off = base + c * CHUNK
        pltpu.sync_copy(src_h.at[pl.ds(off, CHUNK)], idxa_v)
        pltpu.sync_copy(dst_h.at[pl.ds(off, CHUNK)], idxb_v)
        pltpu.sync_copy(rowa_v, shared.at[idxa_v], add=True)
        pltpu.sync_copy(rowb_v, shared.at[idxb_v], add=True)
        return carry

    lax.fori_loop(0, CHUNKS_PER_TILE, body, 0)
    plsc.subcore_barrier()
    pltpu.sync_copy(shared.at[pl.ds(sid * ROWS_PER_SUB, ROWS_PER_SUB)],
                    out_h.at[cid, pl.ds(sid * ROWS_PER_SUB, ROWS_PER_SUB)])


@functools.partial(
    pl.kernel,
    mesh=_MESH,
    compiler_params=_SC_PARAMS,
    out_type=jax.ShapeDtypeStruct((NC, NPAD, DH), jnp.float32),
    scratch_types=[
        pltpu.VMEM((CHUNK,), jnp.int32),
        pltpu.VMEM((CHUNK,), jnp.int32),
        pltpu.VMEM((CHUNK,), jnp.float32),
        pltpu.VMEM((CHUNK, DH), jnp.float32),
        pltpu.VMEM_SHARED((NPAD, DH), jnp.float32),
        pltpu.SemaphoreType.DMA,
    ],
)
def _sc_messages(stab_h, src_h, dst_h, w_h, zeros_h, out_h,
                 idxs_v, idxd_v, w_v, rows_v, shared, sem):
    cid = lax.axis_index("c")
    sid = lax.axis_index("s")
    pltpu.sync_copy(zeros_h.at[pl.ds(sid * ROWS_PER_SUB, ROWS_PER_SUB)],
                    shared.at[pl.ds(sid * ROWS_PER_SUB, ROWS_PER_SUB)])
    plsc.subcore_barrier()

    base = (cid * NS + sid) * EP_TILE

    def body(c, carry):
        off = base + c * CHUNK
        pltpu.sync_copy(src_h.at[pl.ds(off, CHUNK)], idxs_v)
        pltpu.async_copy(stab_h.at[idxs_v], rows_v, sem).wait()
        pltpu.sync_copy(w_h.at[pl.ds(off, CHUNK)], w_v)

        def scale(g, c2):
            wv = w_v[pl.ds(g * DH, DH)]
            for l in range(DH):
                r = g * DH + l
                rows_v[r, :] = rows_v[r, :] * wv[l]
            return c2

        lax.fori_loop(0, CHUNK // DH, scale, 0)
        pltpu.sync_copy(dst_h.at[pl.ds(off, CHUNK)], idxd_v)
        pltpu.sync_copy(rows_v, shared.at[idxd_v], add=True)
        return carry

    lax.fori_loop(0, CHUNKS_PER_TILE, body, 0)
    plsc.subcore_barrier()
    pltpu.sync_copy(shared.at[pl.ds(sid * ROWS_PER_SUB, ROWS_PER_SUB)],
                    out_h.at[cid, pl.ds(sid * ROWS_PER_SUB, ROWS_PER_SUB)])


def _tc_proj1(feat_pad, w1, degtab):
    def body(f_ref, w_ref, deg_ref, s1_ref, dsqo_ref, dsqi_ref):
        dego = deg_ref[0, :, 0:1] + deg_ref[1, :, 0:1]
        degi = deg_ref[0, :, 1:2] + deg_ref[1, :, 1:2]
        dsqo = lax.rsqrt(jnp.maximum(dego, 1.0))
        dsqi = lax.rsqrt(jnp.maximum(degi, 1.0))
        p = jnp.dot(f_ref[...], w_ref[...], preferred_element_type=jnp.float32)
        s1_ref[...] = p * dsqo
        dsqo_ref[...] = dsqo
        dsqi_ref[...] = dsqi

    return pl.pallas_call(
        body,
        out_shape=[
            jax.ShapeDtypeStruct((NPAD, DH), jnp.float32),
            jax.ShapeDtypeStruct((NPAD, 1), jnp.float32),
            jax.ShapeDtypeStruct((NPAD, 1), jnp.float32),
        ],
    )(feat_pad, w1, degtab)


def _tc_layer2_in(agg1, dsqo, dsqi, b1, w2):
    def body(agg_ref, dsqo_ref, dsqi_ref, b1_ref, w2_ref, s2_ref):
        agg = agg_ref[0] + agg_ref[1]
        h = agg * dsqi_ref[...] + b1_ref[...]
        p2 = jnp.dot(h, w2_ref[...], preferred_element_type=jnp.float32)
        s2_ref[...] = p2 * dsqo_ref[...]

    return pl.pallas_call(
        body,
        out_shape=jax.ShapeDtypeStruct((NPAD, DH), jnp.float32),
    )(agg1, dsqo, dsqi, b1, w2)


def _tc_z(agg2, dsqi, b2):
    def body(agg_ref, dsqi_ref, b2_ref, z_ref):
        z_ref[...] = (agg_ref[0] + agg_ref[1]) * dsqi_ref[...] + b2_ref[...]

    return pl.pallas_call(
        body,
        out_shape=jax.ShapeDtypeStruct((NPAD, DH), jnp.float32),
    )(agg2, dsqi, b2)


_BR = 80


def _tc_decoder(z_pad):
    def body(zr_ref, zall_ref, adj_ref):
        full = lax.dot_general(
            zr_ref[...], zall_ref[...],
            (((1,), (1,)), ((), ())),
            preferred_element_type=jnp.float32,
        )
        adj_ref[...] = full[:, :N]

    return pl.pallas_call(
        body,
        grid=(N // _BR,),
        in_specs=[
            pl.BlockSpec((_BR, DH), lambda i: (i, 0)),
            pl.BlockSpec((NPAD, DH), lambda i: (0, 0)),
        ],
        out_specs=pl.BlockSpec((_BR, N), lambda i: (i, 0)),
        out_shape=jax.ShapeDtypeStruct((N, N), jnp.float32),
    )(z_pad, z_pad)


def kernel(features, edge_index, edge_weight, W1, b1, W2, b2):
    src = edge_index[0]
    dst = edge_index[1]
    pad_e = EPAD - E
    src_p = jnp.concatenate([src, jnp.full((pad_e,), N, jnp.int32)])
    dst_p = jnp.concatenate([dst, jnp.full((pad_e,), N, jnp.int32)])
    w_p = jnp.concatenate([edge_weight, jnp.zeros((pad_e,), jnp.float32)])
    feat_pad = jnp.pad(features, ((0, NPAD - N), (0, 0)))
    zeros_tab = jnp.zeros((NPAD, DH), jnp.float32)
    onerows = jnp.tile(
        (jnp.arange(DH)[None, :] == jnp.arange(2)[:, None]
         ).astype(jnp.float32)[:, None, :],
        (1, CHUNK, 1))
    b1r = b1.reshape(1, DH)
    b2r = b2.reshape(1, DH)

    degtab = _sc_degrees(src_p, dst_p, onerows, zeros_tab)
    s1, dsqo, dsqi = _tc_proj1(feat_pad, W1, degtab)
    agg1 = _sc_messages(s1, src_p, dst_p, w_p, zeros_tab)
    s2 = _tc_layer2_in(agg1, dsqo, dsqi, b1r, W2)
    agg2 = _sc_messages(s2, src_p, dst_p, w_p, zeros_tab)
    z_pad = _tc_z(agg2, dsqi, b2r)
    adj = _tc_decoder(z_pad)
    return (adj, z_pad[:N])


# speedup vs baseline: 5.9944x; 1.1082x over previous
import functools

import jax
import jax.numpy as jnp
from jax import lax
from jax.experimental import pallas as pl
from jax.experimental.pallas import tpu as pltpu
from jax.experimental.pallas import tpu_sc as plsc

N = 10000
E = 320000
D_IN = 128
DH = 16

NC = 2
NS = 16
NW = NC * NS

NPAD = 10112
ROWS_PER_SUB = NPAD // NS

CHUNK = 128
CPT = 80
EP_TILE = CPT * CHUNK
EPAD = NW * EP_TILE
TCH = EPAD // CHUNK

_MESH = plsc.VectorSubcoreMesh(core_axis_name="c", subcore_axis_name="s")
_SC_PARAMS = pltpu.CompilerParams(use_tc_tiling_on_sc=False)


@functools.partial(
    pl.kernel,
    mesh=_MESH,
    compiler_params=_SC_PARAMS,
    out_type=jax.ShapeDtypeStruct((NC, NPAD, DH), jnp.float32),
    scratch_types=[
        pltpu.VMEM((CPT, CHUNK), jnp.int32),
        pltpu.VMEM((CPT, CHUNK), jnp.int32),
        pltpu.VMEM((CHUNK, DH), jnp.float32),
        pltpu.VMEM((CHUNK, DH), jnp.float32),
        pltpu.VMEM_SHARED((NPAD, DH), jnp.float32),
        pltpu.SemaphoreType.DMA,
    ],
)
def _sc_degrees(src_h, dst_h, onerows_h, zeros_h, out_h,
                src_v, dst_v, rowa_v, rowb_v, shared, sem):
    cid = lax.axis_index("c")
    sid = lax.axis_index("s")
    tile = cid * NS + sid
    pltpu.sync_copy(src_h.at[pl.ds(tile * CPT, CPT)], src_v)
    pltpu.sync_copy(dst_h.at[pl.ds(tile * CPT, CPT)], dst_v)
    pltpu.sync_copy(onerows_h.at[0], rowa_v)
    pltpu.sync_copy(onerows_h.at[1], rowb_v)
    pltpu.sync_copy(zeros_h.at[pl.ds(sid * ROWS_PER_SUB, ROWS_PER_SUB)],
                    shared.at[pl.ds(sid * ROWS_PER_SUB, ROWS_PER_SUB)])
    plsc.subcore_barrier()

    pltpu.async_copy(rowa_v, shared.at[src_v.at[0]], sem, add=True)
    pltpu.async_copy(rowb_v, shared.at[dst_v.at[0]], sem, add=True)

    def body(c, carry):
        pltpu.async_copy(rowa_v, shared.at[src_v.at[c]], sem, add=True)
        pltpu.async_copy(rowb_v, shared.at[dst_v.at[c]], sem, add=True)
        pltpu.make_async_copy(rowa_v, shared.at[src_v.at[0]], sem).wait()
        pltpu.make_async_copy(rowb_v, shared.at[dst_v.at[0]], sem).wait()
        return carry

    lax.fori_loop(1, CPT, body, 0)
    pltpu.make_async_copy(rowa_v, shared.at[src_v.at[0]], sem).wait()
    pltpu.make_async_copy(rowb_v, shared.at[dst_v.at[0]], sem).wait()
    plsc.subcore_barrier()
    pltpu.sync_copy(shared.at[pl.ds(sid * ROWS_PER_SUB, ROWS_PER_SUB)],
                    out_h.at[cid, pl.ds(sid * ROWS_PER_SUB, ROWS_PER_SUB)])


@functools.partial(
    pl.kernel,
    mesh=_MESH,
    compiler_params=_SC_PARAMS,
    out_type=jax.ShapeDtypeStruct((NC, NPAD, DH), jnp.float32),
    scratch_types=[
        pltpu.VMEM((CPT, CHUNK), jnp.int32),
        pltpu.VMEM((CPT, CHUNK), jnp.int32),
        pltpu.VMEM((CHUNK, DH), jnp.float32),
        pltpu.VMEM((CHUNK, DH), jnp.float32),
        pltpu.VMEM((CHUNK, DH), jnp.float32),
        pltpu.VMEM((CHUNK, DH), jnp.float32),
        pltpu.VMEM_SHARED((NPAD, DH), jnp.float32),
        pltpu.SemaphoreType.DMA,
        pltpu.SemaphoreType.DMA,
        pltpu.SemaphoreType.DMA,
        pltpu.SemaphoreType.DMA,
    ],
)
def _sc_messages(stab_h, wrows_h, src_h, dst_h, zeros_h, out_h,
                 src_v, dst_v, rows0_v, rows1_v, w0_v, w1_v, shared,
                 semr0, semr1, semw0, semw1):
    cid = lax.axis_index("c")
    sid = lax.axis_index("s")
    tile = cid * NS + sid
    pltpu.sync_copy(src_h.at[pl.ds(tile * CPT, CPT)], src_v)
    pltpu.sync_copy(dst_h.at[pl.ds(tile * CPT, CPT)], dst_v)
    pltpu.sync_copy(zeros_h.at[pl.ds(sid * ROWS_PER_SUB, ROWS_PER_SUB)],
                    shared.at[pl.ds(sid * ROWS_PER_SUB, ROWS_PER_SUB)])
    plsc.subcore_barrier()

    ebase = tile * EP_TILE

    def gather(c, rows_v, w_v, semr, semw):
        pltpu.async_copy(stab_h.at[src_v.at[c]], rows_v, semr)
        pltpu.async_copy(wrows_h.at[pl.ds(ebase + c * CHUNK, CHUNK)],
                         w_v, semw)

    def process(c, rows_v, w_v, semr, semw):
        pltpu.make_async_copy(stab_h.at[src_v.at[0]], rows_v, semr).wait()
        pltpu.make_async_copy(wrows_h.at[pl.ds(0, CHUNK)], w_v, semw).wait()

        def scale(g, carry):
            for l in range(DH):
                r = g * DH + l
                rows_v[r, :] = rows_v[r, :] * w_v[r, :]
            return carry

        lax.fori_loop(0, CHUNK // DH, scale, 0)
        pltpu.sync_copy(rows_v, shared.at[dst_v.at[c]], add=True)

    gather(0, rows0_v, w0_v, semr0, semw0)

    def body(i, carry):
        c0 = 2 * i
        gather(c0 + 1, rows1_v, w1_v, semr1, semw1)
        process(c0, rows0_v, w0_v, semr0, semw0)

        @pl.when(i < CPT // 2 - 1)
        def _():
            gather(c0 + 2, rows0_v, w0_v, semr0, semw0)

        process(c0 + 1, rows1_v, w1_v, semr1, semw1)
        return carry

    lax.fori_loop(0, CPT // 2, body, 0)
    plsc.subcore_barrier()
    pltpu.sync_copy(shared.at[pl.ds(sid * ROWS_PER_SUB, ROWS_PER_SUB)],
                    out_h.at[cid, pl.ds(sid * ROWS_PER_SUB, ROWS_PER_SUB)])


def _tc_proj(feat_pad, w1):
    def body(f_ref, w_ref, p_ref):
        p_ref[...] = jnp.dot(f_ref[...], w_ref[...],
                             preferred_element_type=jnp.float32)

    return pl.pallas_call(
        body,
        out_shape=jax.ShapeDtypeStruct((NPAD, DH), jnp.float32),
    )(feat_pad, w1)


def _tc_wrows(w_col):
    def body(w_ref, out_ref):
        out_ref[...] = w_ref[...] * jnp.ones((1, DH), jnp.float32)

    blk = 8192
    return pl.pallas_call(
        body,
        grid=(EPAD // blk,),
        in_specs=[pl.BlockSpec((blk, 1), lambda i: (i, 0))],
        out_specs=pl.BlockSpec((blk, DH), lambda i: (i, 0)),
        out_shape=jax.ShapeDtypeStruct((EPAD, DH), jnp.float32),
    )(w_col)


def _tc_scale1(p1, degtab):
    def body(p_ref, deg_ref, s1_ref, dsqo_ref, dsqi_ref):
        dego = deg_ref[0, :, 0:1] + deg_ref[1, :, 0:1]
        degi = deg_ref[0, :, 1:2] + deg_ref[1, :, 1:2]
        dsqo = lax.rsqrt(jnp.maximum(dego, 1.0))
        dsqi = lax.rsqrt(jnp.maximum(degi, 1.0))
        s1_ref[...] = p_ref[...] * dsqo
        dsqo_ref[...] = dsqo
        dsqi_ref[...] = dsqi

    return pl.pallas_call(
        body,
        out_shape=[
            jax.ShapeDtypeStruct((NPAD, DH), jnp.float32),
            jax.ShapeDtypeStruct((NPAD, 1), jnp.float32),
            jax.ShapeDtypeStruct((NPAD, 1), jnp.float32),
        ],
    )(p1, degtab)


def _tc_layer2_in(agg1, dsqo, dsqi, b1, w2):
    def body(agg_ref, dsqo_ref, dsqi_ref, b1_ref, w2_ref, s2_ref):
        agg = agg_ref[0] + agg_ref[1]
        h = agg * dsqi_ref[...] + b1_ref[...]
        p2 = jnp.dot(h, w2_ref[...], preferred_element_type=jnp.float32)
        s2_ref[...] = p2 * dsqo_ref[...]

    return pl.pallas_call(
        body,
        out_shape=jax.ShapeDtypeStruct((NPAD, DH), jnp.float32),
    )(agg1, dsqo, dsqi, b1, w2)


def _tc_z(agg2, dsqi, b2):
    def body(agg_ref, dsqi_ref, b2_ref, z_ref):
        z_ref[...] = (agg_ref[0] + agg_ref[1]) * dsqi_ref[...] + b2_ref[...]

    return pl.pallas_call(
        body,
        out_shape=jax.ShapeDtypeStruct((NPAD, DH), jnp.float32),
    )(agg2, dsqi, b2)


_BR = 80


def _tc_decoder(z_pad):
    def body(zr_ref, zall_ref, adj_ref):
        full = lax.dot_general(
            zr_ref[...], zall_ref[...],
            (((1,), (1,)), ((), ())),
            preferred_element_type=jnp.float32,
        )
        adj_ref[...] = full[:, :N]

    return pl.pallas_call(
        body,
        grid=(N // _BR,),
        in_specs=[
            pl.BlockSpec((_BR, DH), lambda i: (i, 0)),
            pl.BlockSpec((NPAD, DH), lambda i: (0, 0)),
        ],
        out_specs=pl.BlockSpec((_BR, N), lambda i: (i, 0)),
        out_shape=jax.ShapeDtypeStruct((N, N), jnp.float32),
    )(z_pad, z_pad)


def kernel(features, edge_index, edge_weight, W1, b1, W2, b2):
    src = edge_index[0]
    dst = edge_index[1]
    pad_e = EPAD - E
    src_p = jnp.concatenate([src, jnp.full((pad_e,), N, jnp.int32)])
    dst_p = jnp.concatenate([dst, jnp.full((pad_e,), N, jnp.int32)])
    w_p = jnp.concatenate([edge_weight, jnp.zeros((pad_e,), jnp.float32)])
    src2d = src_p.reshape(TCH, CHUNK)
    dst2d = dst_p.reshape(TCH, CHUNK)
    w_col = w_p.reshape(EPAD, 1)
    feat_pad = jnp.pad(features, ((0, NPAD - N), (0, 0)))
    zeros_tab = jnp.zeros((NPAD, DH), jnp.float32)
    onerows = jnp.tile(
        (jnp.arange(DH)[None, :] == jnp.arange(2)[:, None]
         ).astype(jnp.float32)[:, None, :],
        (1, CHUNK, 1))
    b1r = b1.reshape(1, DH)
    b2r = b2.reshape(1, DH)

    degtab = _sc_degrees(src2d, dst2d, onerows, zeros_tab)
    p1 = _tc_proj(feat_pad, W1)
    wrows = _tc_wrows(w_col)
    s1, dsqo, dsqi = _tc_scale1(p1, degtab)
    agg1 = _sc_messages(s1, wrows, src2d, dst2d, zeros_tab)
    s2 = _tc_layer2_in(agg1, dsqo, dsqi, b1r, W2)
    agg2 = _sc_messages(s2, wrows, src2d, dst2d, zeros_tab)
    z_pad = _tc_z(agg2, dsqi, b2r)
    adj = _tc_decoder(z_pad)
    return (adj, z_pad[:N])


# speedup vs baseline: 9.1175x; 1.5210x over previous
import functools

import jax
import jax.numpy as jnp
from jax import lax
from jax.experimental import pallas as pl
from jax.experimental.pallas import tpu as pltpu
from jax.experimental.pallas import tpu_sc as plsc

N = 10000
E = 320000
D_IN = 128
DH = 16

NC = 2
NS = 16
NW = NC * NS

NPAD = 10112
ROWS_PER_SUB = NPAD // NS

CHUNK = 128
CPT = 80
EP_TILE = CPT * CHUNK
EPAD = NW * EP_TILE
TCH = EPAD // CHUNK

_MESH = plsc.VectorSubcoreMesh(core_axis_name="c", subcore_axis_name="s")
_SC_PARAMS = pltpu.CompilerParams(use_tc_tiling_on_sc=False)


@functools.partial(
    pl.kernel,
    mesh=_MESH,
    compiler_params=_SC_PARAMS,
    out_type=jax.ShapeDtypeStruct((NC, NPAD, DH), jnp.float32),
    scratch_types=[
        pltpu.VMEM((CPT, CHUNK), jnp.int32),
        pltpu.VMEM((CPT, CHUNK), jnp.int32),
        pltpu.VMEM((CHUNK, DH), jnp.float32),
        pltpu.VMEM((CHUNK, DH), jnp.float32),
        pltpu.VMEM_SHARED((NPAD, DH), jnp.float32),
        pltpu.SemaphoreType.DMA,
    ],
)
def _sc_degrees(src_h, dst_h, onerows_h, zeros_h, out_h,
                src_v, dst_v, rowa_v, rowb_v, shared, sem):
    cid = lax.axis_index("c")
    sid = lax.axis_index("s")
    tile = cid * NS + sid
    pltpu.sync_copy(src_h.at[pl.ds(tile * CPT, CPT)], src_v)
    pltpu.sync_copy(dst_h.at[pl.ds(tile * CPT, CPT)], dst_v)
    pltpu.sync_copy(onerows_h.at[0], rowa_v)
    pltpu.sync_copy(onerows_h.at[1], rowb_v)
    pltpu.sync_copy(zeros_h.at[pl.ds(sid * ROWS_PER_SUB, ROWS_PER_SUB)],
                    shared.at[pl.ds(sid * ROWS_PER_SUB, ROWS_PER_SUB)])
    plsc.subcore_barrier()

    def issue(c):
        pltpu.async_copy(rowa_v, shared.at[src_v.at[c]], sem, add=True)
        pltpu.async_copy(rowb_v, shared.at[dst_v.at[c]], sem, add=True)

    def drain():
        pltpu.make_async_copy(rowa_v, shared.at[src_v.at[0]], sem).wait()
        pltpu.make_async_copy(rowb_v, shared.at[dst_v.at[0]], sem).wait()

    issue(0)
    issue(1)
    issue(2)

    def body(c, carry):
        issue(c)
        drain()
        return carry

    lax.fori_loop(3, CPT, body, 0)
    drain()
    drain()
    drain()
    plsc.subcore_barrier()
    pltpu.sync_copy(shared.at[pl.ds(sid * ROWS_PER_SUB, ROWS_PER_SUB)],
                    out_h.at[cid, pl.ds(sid * ROWS_PER_SUB, ROWS_PER_SUB)])


@functools.partial(
    pl.kernel,
    mesh=_MESH,
    compiler_params=_SC_PARAMS,
    out_type=jax.ShapeDtypeStruct((NC, NPAD, DH), jnp.float32),
    scratch_types=[
        pltpu.VMEM((CPT, CHUNK), jnp.int32),
        pltpu.VMEM((CPT, CHUNK), jnp.int32),
        pltpu.VMEM((CPT, CHUNK), jnp.float32),
        pltpu.VMEM((CHUNK, DH), jnp.float32),
        pltpu.VMEM((CHUNK, DH), jnp.float32),
        pltpu.VMEM((CHUNK, DH), jnp.float32),
        pltpu.VMEM((CHUNK, DH), jnp.float32),
        pltpu.VMEM_SHARED((NPAD, DH), jnp.float32),
        pltpu.SemaphoreType.DMA,
        pltpu.SemaphoreType.DMA,
        pltpu.SemaphoreType.DMA,
        pltpu.SemaphoreType.DMA,
        pltpu.SemaphoreType.DMA,
        pltpu.SemaphoreType.DMA,
        pltpu.SemaphoreType.DMA,
        pltpu.SemaphoreType.DMA,
    ],
)
def _sc_messages(stab_h, src_h, dst_h, w_h, zeros_h, out_h,
                 src_v, dst_v, w_v, b0, b1, b2, b3, shared,
                 sg0, sg1, sg2, sg3, ss0, ss1, ss2, ss3):
    cid = lax.axis_index("c")
    sid = lax.axis_index("s")
    tile = cid * NS + sid
    pltpu.sync_copy(src_h.at[pl.ds(tile * CPT, CPT)], src_v)
    pltpu.sync_copy(dst_h.at[pl.ds(tile * CPT, CPT)], dst_v)
    pltpu.sync_copy(w_h.at[pl.ds(tile * CPT, CPT)], w_v)
    pltpu.sync_copy(zeros_h.at[pl.ds(sid * ROWS_PER_SUB, ROWS_PER_SUB)],
                    shared.at[pl.ds(sid * ROWS_PER_SUB, ROWS_PER_SUB)])
    plsc.subcore_barrier()

    bufs = (b0, b1, b2, b3)
    gsems = (sg0, sg1, sg2, sg3)
    ssems = (ss0, ss1, ss2, ss3)

    def gather(c, k):
        pltpu.async_copy(stab_h.at[src_v.at[c]], bufs[k], gsems[k])

    def wait_gather(k):
        pltpu.make_async_copy(stab_h.at[src_v.at[0]], bufs[k],
                              gsems[k]).wait()

    def scatter(c, k):
        pltpu.async_copy(bufs[k], shared.at[dst_v.at[c]], ssems[k], add=True)

    def wait_scatter(k):
        pltpu.make_async_copy(bufs[k], shared.at[dst_v.at[0]],
                              ssems[k]).wait()

    def scale(k, c):
        rows_v = bufs[k]

        def group(g, carry):
            wv = w_v[c, pl.ds(g * DH, DH)]
            for l in range(DH):
                r = g * DH + l
                rows_v[r, :] = rows_v[r, :] * wv[l]
            return carry

        lax.fori_loop(0, CHUNK // DH, group, 0)

    gather(0, 0)
    gather(1, 1)
    gather(2, 2)

    def body(i, carry):
        for k in range(4):
            c = 4 * i + k

            @pl.when((c >= 1) & (c + 3 < CPT))
            def _():
                wait_scatter((k + 3) % 4)

            @pl.when(c + 3 < CPT)
            def _():
                gather(c + 3, (k + 3) % 4)

            wait_gather(k)
            scale(k, c)
            scatter(c, k)
        return carry

    lax.fori_loop(0, CPT // 4, body, 0)
    for k in range(4):
        wait_scatter(k)
    plsc.subcore_barrier()
    pltpu.sync_copy(shared.at[pl.ds(sid * ROWS_PER_SUB, ROWS_PER_SUB)],
                    out_h.at[cid, pl.ds(sid * ROWS_PER_SUB, ROWS_PER_SUB)])


def _tc_proj(feat_pad, w1):
    def body(f_ref, w_ref, p_ref):
        p_ref[...] = jnp.dot(f_ref[...], w_ref[...],
                             preferred_element_type=jnp.float32)

    return pl.pallas_call(
        body,
        out_shape=jax.ShapeDtypeStruct((NPAD, DH), jnp.float32),
    )(feat_pad, w1)


def _tc_scale1(p1, degtab):
    def body(p_ref, deg_ref, s1_ref, dsqo_ref, dsqi_ref):
        dego = deg_ref[0, :, 0:1] + deg_ref[1, :, 0:1]
        degi = deg_ref[0, :, 1:2] + deg_ref[1, :, 1:2]
        dsqo = lax.rsqrt(jnp.maximum(dego, 1.0))
        dsqi = lax.rsqrt(jnp.maximum(degi, 1.0))
        s1_ref[...] = p_ref[...] * dsqo
        dsqo_ref[...] = dsqo
        dsqi_ref[...] = dsqi

    return pl.pallas_call(
        body,
        out_shape=[
            jax.ShapeDtypeStruct((NPAD, DH), jnp.float32),
            jax.ShapeDtypeStruct((NPAD, 1), jnp.float32),
            jax.ShapeDtypeStruct((NPAD, 1), jnp.float32),
        ],
    )(p1, degtab)


def _tc_layer2_in(agg1, dsqo, dsqi, b1, w2):
    def body(agg_ref, dsqo_ref, dsqi_ref, b1_ref, w2_ref, s2_ref):
        agg = agg_ref[0] + agg_ref[1]
        h = agg * dsqi_ref[...] + b1_ref[...]
        p2 = jnp.dot(h, w2_ref[...], preferred_element_type=jnp.float32)
        s2_ref[...] = p2 * dsqo_ref[...]

    return pl.pallas_call(
        body,
        out_shape=jax.ShapeDtypeStruct((NPAD, DH), jnp.float32),
    )(agg1, dsqo, dsqi, b1, w2)


def _tc_z(agg2, dsqi, b2):
    def body(agg_ref, dsqi_ref, b2_ref, z_ref):
        z_ref[...] = (agg_ref[0] + agg_ref[1]) * dsqi_ref[...] + b2_ref[...]

    return pl.pallas_call(
        body,
        out_shape=jax.ShapeDtypeStruct((NPAD, DH), jnp.float32),
    )(agg2, dsqi, b2)


_BR = 80


def _tc_decoder(z_pad):
    def body(zr_ref, zall_ref, adj_ref):
        full = lax.dot_general(
            zr_ref[...], zall_ref[...],
            (((1,), (1,)), ((), ())),
            preferred_element_type=jnp.float32,
        )
        adj_ref[...] = full[:, :N]

    return pl.pallas_call(
        body,
        grid=(N // _BR,),
        in_specs=[
            pl.BlockSpec((_BR, DH), lambda i: (i, 0)),
            pl.BlockSpec((NPAD, DH), lambda i: (0, 0)),
        ],
        out_specs=pl.BlockSpec((_BR, N), lambda i: (i, 0)),
        out_shape=jax.ShapeDtypeStruct((N, N), jnp.float32),
    )(z_pad, z_pad)


def kernel(features, edge_index, edge_weight, W1, b1, W2, b2):
    src = edge_index[0]
    dst = edge_index[1]
    pad_e = EPAD - E
    src_p = jnp.concatenate([src, jnp.full((pad_e,), N, jnp.int32)])
    dst_p = jnp.concatenate([dst, jnp.full((pad_e,), N, jnp.int32)])
    w_p = jnp.concatenate([edge_weight, jnp.zeros((pad_e,), jnp.float32)])
    src2d = src_p.reshape(TCH, CHUNK)
    dst2d = dst_p.reshape(TCH, CHUNK)
    w2d = w_p.reshape(TCH, CHUNK)
    feat_pad = jnp.pad(features, ((0, NPAD - N), (0, 0)))
    zeros_tab = jnp.zeros((NPAD, DH), jnp.float32)
    onerows = jnp.tile(
        (jnp.arange(DH)[None, :] == jnp.arange(2)[:, None]
         ).astype(jnp.float32)[:, None, :],
        (1, CHUNK, 1))
    b1r = b1.reshape(1, DH)
    b2r = b2.reshape(1, DH)

    degtab = _sc_degrees(src2d, dst2d, onerows, zeros_tab)
    p1 = _tc_proj(feat_pad, W1)
    s1, dsqo, dsqi = _tc_scale1(p1, degtab)
    agg1 = _sc_messages(s1, src2d, dst2d, w2d, zeros_tab)
    s2 = _tc_layer2_in(agg1, dsqo, dsqi, b1r, W2)
    agg2 = _sc_messages(s2, src2d, dst2d, w2d, zeros_tab)
    z_pad = _tc_z(agg2, dsqi, b2r)
    adj = _tc_decoder(z_pad)
    return (adj, z_pad[:N])


# speedup vs baseline: 11.6003x; 1.2723x over previous
import functools

import jax
import jax.numpy as jnp
from jax import lax
from jax.experimental import pallas as pl
from jax.experimental.pallas import tpu as pltpu
from jax.experimental.pallas import tpu_sc as plsc

N = 10000
E = 320000
D_IN = 128
DH = 16

NC = 2
NS = 16
NW = NC * NS

NPAD = 10112
ROWS_PER_SUB = NPAD // NS

CHUNK = 128
CPT = 80
EP_TILE = CPT * CHUNK
EPAD = NW * EP_TILE
TCH = EPAD // CHUNK

_MESH = plsc.VectorSubcoreMesh(core_axis_name="c", subcore_axis_name="s")
_SC_PARAMS = pltpu.CompilerParams(use_tc_tiling_on_sc=False)


@functools.partial(
    pl.kernel,
    mesh=_MESH,
    compiler_params=_SC_PARAMS,
    out_type=jax.ShapeDtypeStruct((NC, NPAD, DH), jnp.float32),
    scratch_types=[
        pltpu.VMEM((CPT, CHUNK), jnp.int32),
        pltpu.VMEM((CPT, CHUNK), jnp.int32),
        pltpu.VMEM((CHUNK, DH), jnp.float32),
        pltpu.VMEM((CHUNK, DH), jnp.float32),
        pltpu.VMEM_SHARED((NPAD, DH), jnp.float32),
        pltpu.SemaphoreType.DMA,
    ],
)
def _sc_degrees(src_h, dst_h, onerows_h, zeros_h, out_h,
                src_v, dst_v, rowa_v, rowb_v, shared, sem):
    cid = lax.axis_index("c")
    sid = lax.axis_index("s")
    tile = cid * NS + sid
    pltpu.sync_copy(src_h.at[pl.ds(tile * CPT, CPT)], src_v)
    pltpu.sync_copy(dst_h.at[pl.ds(tile * CPT, CPT)], dst_v)
    pltpu.sync_copy(onerows_h.at[0], rowa_v)
    pltpu.sync_copy(onerows_h.at[1], rowb_v)
    pltpu.sync_copy(zeros_h.at[pl.ds(sid * ROWS_PER_SUB, ROWS_PER_SUB)],
                    shared.at[pl.ds(sid * ROWS_PER_SUB, ROWS_PER_SUB)])
    plsc.subcore_barrier()

    def issue(c):
        pltpu.async_copy(rowa_v, shared.at[src_v.at[c]], sem, add=True)
        pltpu.async_copy(rowb_v, shared.at[dst_v.at[c]], sem, add=True)

    def drain():
        pltpu.make_async_copy(rowa_v, shared.at[src_v.at[0]], sem).wait()
        pltpu.make_async_copy(rowb_v, shared.at[dst_v.at[0]], sem).wait()

    issue(0)
    issue(1)
    issue(2)

    def body(c, carry):
        issue(c)
        drain()
        return carry

    lax.fori_loop(3, CPT, body, 0)
    drain()
    drain()
    drain()
    plsc.subcore_barrier()
    pltpu.sync_copy(shared.at[pl.ds(sid * ROWS_PER_SUB, ROWS_PER_SUB)],
                    out_h.at[cid, pl.ds(sid * ROWS_PER_SUB, ROWS_PER_SUB)])


@functools.partial(
    pl.kernel,
    mesh=_MESH,
    compiler_params=_SC_PARAMS,
    out_type=jax.ShapeDtypeStruct((NC, NPAD, DH), jnp.float32),
    scratch_types=[
        pltpu.VMEM((CPT, CHUNK), jnp.int32),
        pltpu.VMEM((CPT, CHUNK), jnp.int32),
        pltpu.VMEM((CPT, CHUNK), jnp.float32),
        pltpu.VMEM((CHUNK, DH), jnp.float32),
        pltpu.VMEM((CHUNK, DH), jnp.float32),
        pltpu.VMEM((CHUNK, DH), jnp.float32),
        pltpu.VMEM((CHUNK, DH), jnp.float32),
        pltpu.VMEM_SHARED((NPAD, DH), jnp.float32),
        pltpu.SemaphoreType.DMA,
        pltpu.SemaphoreType.DMA,
        pltpu.SemaphoreType.DMA,
        pltpu.SemaphoreType.DMA,
        pltpu.SemaphoreType.DMA,
        pltpu.SemaphoreType.DMA,
        pltpu.SemaphoreType.DMA,
        pltpu.SemaphoreType.DMA,
    ],
)
def _sc_messages(stab_h, src_h, dst_h, w_h, zeros_h, out_h,
                 src_v, dst_v, w_v, b0, b1, b2, b3, shared,
                 sg0, sg1, sg2, sg3, ss0, ss1, ss2, ss3):
    cid = lax.axis_index("c")
    sid = lax.axis_index("s")
    tile = cid * NS + sid
    pltpu.sync_copy(src_h.at[pl.ds(tile * CPT, CPT)], src_v)
    pltpu.sync_copy(dst_h.at[pl.ds(tile * CPT, CPT)], dst_v)
    pltpu.sync_copy(w_h.at[pl.ds(tile * CPT, CPT)], w_v)
    pltpu.sync_copy(zeros_h.at[pl.ds(sid * ROWS_PER_SUB, ROWS_PER_SUB)],
                    shared.at[pl.ds(sid * ROWS_PER_SUB, ROWS_PER_SUB)])
    plsc.subcore_barrier()

    bufs = (b0, b1, b2, b3)
    gsems = (sg0, sg1, sg2, sg3)
    ssems = (ss0, ss1, ss2, ss3)

    def gather(c, k):
        pltpu.async_copy(stab_h.at[src_v.at[c]], bufs[k], gsems[k])

    def wait_gather(k):
        pltpu.make_async_copy(stab_h.at[src_v.at[0]], bufs[k],
                              gsems[k]).wait()

    def scatter(c, k):
        pltpu.async_copy(bufs[k], shared.at[dst_v.at[c]], ssems[k], add=True)

    def wait_scatter(k):
        pltpu.make_async_copy(bufs[k], shared.at[dst_v.at[0]],
                              ssems[k]).wait()

    def scale(k, c):
        rows_v = bufs[k]

        def group(g, carry):
            wv = w_v[c, pl.ds(g * DH, DH)]
            for l in range(DH):
                r = g * DH + l
                rows_v[r, :] = rows_v[r, :] * wv[l]
            return carry

        lax.fori_loop(0, CHUNK // DH, group, 0)

    gather(0, 0)
    gather(1, 1)
    gather(2, 2)

    def body(i, carry):
        for k in range(4):
            c = 4 * i + k

            @pl.when((c >= 1) & (c + 3 < CPT))
            def _():
                wait_scatter((k + 3) % 4)

            @pl.when(c + 3 < CPT)
            def _():
                gather(c + 3, (k + 3) % 4)

            wait_gather(k)
            scale(k, c)
            scatter(c, k)
        return carry

    lax.fori_loop(0, CPT // 4, body, 0)
    for k in range(4):
        wait_scatter(k)
    plsc.subcore_barrier()
    pltpu.sync_copy(shared.at[pl.ds(sid * ROWS_PER_SUB, ROWS_PER_SUB)],
                    out_h.at[cid, pl.ds(sid * ROWS_PER_SUB, ROWS_PER_SUB)])


def _tc_proj(feat_pad, w1):
    def body(f_ref, w_ref, p_ref):
        p_ref[...] = jnp.dot(f_ref[...], w_ref[...],
                             preferred_element_type=jnp.float32)

    return pl.pallas_call(
        body,
        out_shape=jax.ShapeDtypeStruct((NPAD, DH), jnp.float32),
    )(feat_pad, w1)


def _tc_scale1(p1, degtab):
    def body(p_ref, deg_ref, s1_ref, dsqo_ref, dsqi_ref):
        dego = deg_ref[0, :, 0:1] + deg_ref[1, :, 0:1]
        degi = deg_ref[0, :, 1:2] + deg_ref[1, :, 1:2]
        dsqo = lax.rsqrt(jnp.maximum(dego, 1.0))
        dsqi = lax.rsqrt(jnp.maximum(degi, 1.0))
        s1_ref[...] = p_ref[...] * dsqo
        dsqo_ref[...] = dsqo
        dsqi_ref[...] = dsqi

    return pl.pallas_call(
        body,
        out_shape=[
            jax.ShapeDtypeStruct((NPAD, DH), jnp.float32),
            jax.ShapeDtypeStruct((NPAD, 1), jnp.float32),
            jax.ShapeDtypeStruct((NPAD, 1), jnp.float32),
        ],
    )(p1, degtab)


def _tc_layer2_in(agg1, dsqo, dsqi, b1, w2):
    def body(agg_ref, dsqo_ref, dsqi_ref, b1_ref, w2_ref, s2_ref):
        agg = agg_ref[0] + agg_ref[1]
        h = agg * dsqi_ref[...] + b1_ref[...]
        p2 = jnp.dot(h, w2_ref[...], preferred_element_type=jnp.float32)
        s2_ref[...] = p2 * dsqo_ref[...]

    return pl.pallas_call(
        body,
        out_shape=jax.ShapeDtypeStruct((NPAD, DH), jnp.float32),
    )(agg1, dsqo, dsqi, b1, w2)


def _tc_z(agg2, dsqi, b2):
    def body(agg_ref, dsqi_ref, b2_ref, z_ref):
        z_ref[...] = (agg_ref[0] + agg_ref[1]) * dsqi_ref[...] + b2_ref[...]

    return pl.pallas_call(
        body,
        out_shape=jax.ShapeDtypeStruct((NPAD, DH), jnp.float32),
    )(agg2, dsqi, b2)


_BR = 80


def _tc_decoder(z_pad):
    def body(zr_ref, zall_ref, adj_ref):
        full = lax.dot_general(
            zr_ref[...], zall_ref[...],
            (((1,), (1,)), ((), ())),
            preferred_element_type=jnp.float32,
        )
        adj_ref[...] = full[:, :N]

    return pl.pallas_call(
        body,
        grid=(N // _BR,),
        in_specs=[
            pl.BlockSpec((_BR, DH), lambda i: (i, 0)),
            pl.BlockSpec((NPAD, DH), lambda i: (0, 0)),
        ],
        out_specs=pl.BlockSpec((_BR, N), lambda i: (i, 0)),
        out_shape=jax.ShapeDtypeStruct((N, N), jnp.float32),
    )(z_pad, z_pad)


def kernel(features, edge_index, edge_weight, W1, b1, W2, b2):
    src = edge_index[0]
    dst = edge_index[1]
    pad_e = EPAD - E
    pad_nodes = N + (jnp.arange(pad_e, dtype=jnp.int32) % (NPAD - N))
    src_p = jnp.concatenate([src, pad_nodes])
    dst_p = jnp.concatenate([dst, pad_nodes])
    w_p = jnp.concatenate([edge_weight, jnp.zeros((pad_e,), jnp.float32)])
    src2d = src_p.reshape(TCH, CHUNK)
    dst2d = dst_p.reshape(TCH, CHUNK)
    w2d = w_p.reshape(TCH, CHUNK)
    feat_pad = jnp.pad(features, ((0, NPAD - N), (0, 0)))
    zeros_tab = jnp.zeros((NPAD, DH), jnp.float32)
    onerows = jnp.tile(
        (jnp.arange(DH)[None, :] == jnp.arange(2)[:, None]
         ).astype(jnp.float32)[:, None, :],
        (1, CHUNK, 1))
    b1r = b1.reshape(1, DH)
    b2r = b2.reshape(1, DH)

    degtab = _sc_degrees(src2d, dst2d, onerows, zeros_tab)
    p1 = _tc_proj(feat_pad, W1)
    s1, dsqo, dsqi = _tc_scale1(p1, degtab)
    agg1 = _sc_messages(s1, src2d, dst2d, w2d, zeros_tab)
    s2 = _tc_layer2_in(agg1, dsqo, dsqi, b1r, W2)
    agg2 = _sc_messages(s2, src2d, dst2d, w2d, zeros_tab)
    z_pad = _tc_z(agg2, dsqi, b2r)
    adj = _tc_decoder(z_pad)
    return (adj, z_pad[:N])


# speedup vs baseline: 13.3739x; 1.1529x over previous
import functools

import jax
import jax.numpy as jnp
from jax import lax
from jax.experimental import pallas as pl
from jax.experimental.pallas import tpu as pltpu
from jax.experimental.pallas import tpu_sc as plsc

N = 10000
E = 320000
D_IN = 128
DH = 16

NC = 2
NS = 16
NW = NC * NS

NPAD = 10112
ROWS_PER_SUB = NPAD // NS

CHUNK = 128
CPT = 80
EP_TILE = CPT * CHUNK
EPAD = NW * EP_TILE
TCH = EPAD // CHUNK

_MESH = plsc.VectorSubcoreMesh(core_axis_name="c", subcore_axis_name="s")
_SC_PARAMS = pltpu.CompilerParams(use_tc_tiling_on_sc=False)


@functools.partial(
    pl.kernel,
    mesh=_MESH,
    compiler_params=_SC_PARAMS,
    out_type=jax.ShapeDtypeStruct((NC, NPAD, DH), jnp.float32),
    scratch_types=[
        pltpu.VMEM((CPT, CHUNK), jnp.int32),
        pltpu.VMEM((CPT, CHUNK), jnp.int32),
        pltpu.VMEM((CHUNK, DH), jnp.float32),
        pltpu.VMEM((CHUNK, DH), jnp.float32),
        pltpu.VMEM_SHARED((NPAD, DH), jnp.float32),
        pltpu.SemaphoreType.DMA,
    ],
)
def _sc_degrees(src_h, dst_h, onerows_h, zeros_h, out_h,
                src_v, dst_v, rowa_v, rowb_v, shared, sem):
    cid = lax.axis_index("c")
    sid = lax.axis_index("s")
    tile = cid * NS + sid
    pltpu.sync_copy(src_h.at[pl.ds(tile * CPT, CPT)], src_v)
    pltpu.sync_copy(dst_h.at[pl.ds(tile * CPT, CPT)], dst_v)
    pltpu.sync_copy(onerows_h.at[0], rowa_v)
    pltpu.sync_copy(onerows_h.at[1], rowb_v)
    pltpu.sync_copy(zeros_h.at[pl.ds(sid * ROWS_PER_SUB, ROWS_PER_SUB)],
                    shared.at[pl.ds(sid * ROWS_PER_SUB, ROWS_PER_SUB)])
    plsc.subcore_barrier()

    def issue(c):
        pltpu.async_copy(rowa_v, shared.at[src_v.at[c]], sem, add=True)
        pltpu.async_copy(rowb_v, shared.at[dst_v.at[c]], sem, add=True)

    def drain():
        pltpu.make_async_copy(rowa_v, shared.at[src_v.at[0]], sem).wait()
        pltpu.make_async_copy(rowb_v, shared.at[dst_v.at[0]], sem).wait()

    issue(0)
    issue(1)
    issue(2)

    def body(c, carry):
        issue(c)
        drain()
        return carry

    lax.fori_loop(3, CPT, body, 0)
    drain()
    drain()
    drain()
    plsc.subcore_barrier()
    pltpu.sync_copy(shared.at[pl.ds(sid * ROWS_PER_SUB, ROWS_PER_SUB)],
                    out_h.at[cid, pl.ds(sid * ROWS_PER_SUB, ROWS_PER_SUB)])


@functools.partial(
    pl.kernel,
    mesh=_MESH,
    compiler_params=_SC_PARAMS,
    out_type=jax.ShapeDtypeStruct((NC, NPAD, DH), jnp.float32),
    scratch_types=[
        pltpu.VMEM((CPT, CHUNK), jnp.int32),
        pltpu.VMEM((CPT, CHUNK), jnp.int32),
        pltpu.VMEM((CPT, CHUNK), jnp.float32),
        pltpu.VMEM((CHUNK, DH), jnp.float32),
        pltpu.VMEM((CHUNK, DH), jnp.float32),
        pltpu.VMEM((CHUNK, DH), jnp.float32),
        pltpu.VMEM((CHUNK, DH), jnp.float32),
        pltpu.VMEM_SHARED((NPAD, DH), jnp.float32),
        pltpu.SemaphoreType.DMA,
        pltpu.SemaphoreType.DMA,
        pltpu.SemaphoreType.DMA,
        pltpu.SemaphoreType.DMA,
        pltpu.SemaphoreType.DMA,
        pltpu.SemaphoreType.DMA,
        pltpu.SemaphoreType.DMA,
        pltpu.SemaphoreType.DMA,
    ],
)
def _sc_messages(stab_h, src_h, dst_h, w_h, zeros_h, out_h,
                 src_v, dst_v, w_v, b0, b1, b2, b3, shared,
                 sg0, sg1, sg2, sg3, ss0, ss1, ss2, ss3):
    cid = lax.axis_index("c")
    sid = lax.axis_index("s")
    tile = cid * NS + sid
    pltpu.sync_copy(src_h.at[pl.ds(tile * CPT, CPT)], src_v)
    pltpu.sync_copy(dst_h.at[pl.ds(tile * CPT, CPT)], dst_v)
    pltpu.sync_copy(w_h.at[pl.ds(tile * CPT, CPT)], w_v)
    pltpu.sync_copy(zeros_h.at[pl.ds(sid * ROWS_PER_SUB, ROWS_PER_SUB)],
                    shared.at[pl.ds(sid * ROWS_PER_SUB, ROWS_PER_SUB)])
    plsc.subcore_barrier()

    bufs = (b0, b1, b2, b3)
    gsems = (sg0, sg1, sg2, sg3)
    ssems = (ss0, ss1, ss2, ss3)

    def gather(c, k):
        pltpu.async_copy(stab_h.at[src_v.at[c]], bufs[k], gsems[k])

    def wait_gather(k):
        pltpu.make_async_copy(stab_h.at[src_v.at[0]], bufs[k],
                              gsems[k]).wait()

    def scatter(c, k):
        pltpu.async_copy(bufs[k], shared.at[dst_v.at[c]], ssems[k], add=True)

    def wait_scatter(k):
        pltpu.make_async_copy(bufs[k], shared.at[dst_v.at[0]],
                              ssems[k]).wait()

    def scale(k, c):
        rows_v = bufs[k]

        def group(g, carry):
            wv = w_v[c, pl.ds(g * DH, DH)]
            for l in range(DH):
                r = g * DH + l
                rows_v[r, :] = rows_v[r, :] * wv[l]
            return carry

        lax.fori_loop(0, CHUNK // DH, group, 0)

    gather(0, 0)
    gather(1, 1)
    gather(2, 2)

    def body(i, carry):
        for k in range(4):
            c = 4 * i + k

            @pl.when((c >= 1) & (c + 3 < CPT))
            def _():
                wait_scatter((k + 3) % 4)

            @pl.when(c + 3 < CPT)
            def _():
                gather(c + 3, (k + 3) % 4)

            wait_gather(k)
            scale(k, c)
            scatter(c, k)
        return carry

    lax.fori_loop(0, CPT // 4, body, 0)
    for k in range(4):
        wait_scatter(k)
    plsc.subcore_barrier()
    pltpu.sync_copy(shared.at[pl.ds(sid * ROWS_PER_SUB, ROWS_PER_SUB)],
                    out_h.at[cid, pl.ds(sid * ROWS_PER_SUB, ROWS_PER_SUB)])


def _tc_proj(feat_pad, w1):
    def body(f_ref, w_ref, p_ref):
        p_ref[...] = jnp.dot(f_ref[...], w_ref[...],
                             preferred_element_type=jnp.float32)

    return pl.pallas_call(
        body,
        out_shape=jax.ShapeDtypeStruct((NPAD, DH), jnp.float32),
    )(feat_pad, w1)


def _tc_scale1(p1, degtab):
    def body(p_ref, deg_ref, s1_ref, dsqo_ref, dsqi_ref):
        dego = deg_ref[0, :, 0:1] + deg_ref[1, :, 0:1]
        degi = deg_ref[0, :, 1:2] + deg_ref[1, :, 1:2]
        dsqo = lax.rsqrt(jnp.maximum(dego, 1.0))
        dsqi = lax.rsqrt(jnp.maximum(degi, 1.0))
        s1_ref[...] = p_ref[...] * dsqo
        dsqo_ref[...] = dsqo
        dsqi_ref[...] = dsqi

    return pl.pallas_call(
        body,
        out_shape=[
            jax.ShapeDtypeStruct((NPAD, DH), jnp.float32),
            jax.ShapeDtypeStruct((NPAD, 1), jnp.float32),
            jax.ShapeDtypeStruct((NPAD, 1), jnp.float32),
        ],
    )(p1, degtab)


def _tc_layer2_in(agg1, dsqo, dsqi, b1, w2):
    def body(agg_ref, dsqo_ref, dsqi_ref, b1_ref, w2_ref, s2_ref):
        agg = agg_ref[0] + agg_ref[1]
        h = agg * dsqi_ref[...] + b1_ref[...]
        p2 = jnp.dot(h, w2_ref[...], preferred_element_type=jnp.float32)
        s2_ref[...] = p2 * dsqo_ref[...]

    return pl.pallas_call(
        body,
        out_shape=jax.ShapeDtypeStruct((NPAD, DH), jnp.float32),
    )(agg1, dsqo, dsqi, b1, w2)


def _tc_z(agg2, dsqi, b2):
    def body(agg_ref, dsqi_ref, b2_ref, z_ref):
        z_ref[...] = (agg_ref[0] + agg_ref[1]) * dsqi_ref[...] + b2_ref[...]

    return pl.pallas_call(
        body,
        out_shape=jax.ShapeDtypeStruct((NPAD, DH), jnp.float32),
    )(agg2, dsqi, b2)


_BR = 200


def _tc_decoder(z_pad):
    def body(zr_ref, zall_ref, adj_ref):
        full = lax.dot_general(
            zr_ref[...], zall_ref[...],
            (((1,), (1,)), ((), ())),
            preferred_element_type=jnp.float32,
        )
        adj_ref[...] = full[:, :N]

    return pl.pallas_call(
        body,
        grid=(N // _BR,),
        in_specs=[
            pl.BlockSpec((_BR, DH), lambda i: (i, 0)),
            pl.BlockSpec((NPAD, DH), lambda i: (0, 0)),
        ],
        out_specs=pl.BlockSpec((_BR, N), lambda i: (i, 0)),
        out_shape=jax.ShapeDtypeStruct((N, N), jnp.float32),
    )(z_pad, z_pad)


def kernel(features, edge_index, edge_weight, W1, b1, W2, b2):
    src = edge_index[0]
    dst = edge_index[1]
    pad_e = EPAD - E
    pad_nodes = N + (jnp.arange(pad_e, dtype=jnp.int32) % (NPAD - N))
    src_p = jnp.concatenate([src, pad_nodes])
    dst_p = jnp.concatenate([dst, pad_nodes])
    w_p = jnp.concatenate([edge_weight, jnp.zeros((pad_e,), jnp.float32)])
    src2d = src_p.reshape(TCH, CHUNK)
    dst2d = dst_p.reshape(TCH, CHUNK)
    w2d = w_p.reshape(TCH, CHUNK)
    feat_pad = jnp.pad(features, ((0, NPAD - N), (0, 0)))
    zeros_tab = jnp.zeros((NPAD, DH), jnp.float32)
    onerows = jnp.tile(
        (jnp.arange(DH)[None, :] == jnp.arange(2)[:, None]
         ).astype(jnp.float32)[:, None, :],
        (1, CHUNK, 1))
    b1r = b1.reshape(1, DH)
    b2r = b2.reshape(1, DH)

    degtab = _sc_degrees(src2d, dst2d, onerows, zeros_tab)
    p1 = _tc_proj(feat_pad, W1)
    s1, dsqo, dsqi = _tc_scale1(p1, degtab)
    agg1 = _sc_messages(s1, src2d, dst2d, w2d, zeros_tab)
    s2 = _tc_layer2_in(agg1, dsqo, dsqi, b1r, W2)
    agg2 = _sc_messages(s2, src2d, dst2d, w2d, zeros_tab)
    z_pad = _tc_z(agg2, dsqi, b2r)
    adj = _tc_decoder(z_pad)
    return (adj, z_pad[:N])
